# baseline (speedup 1.0000x reference)
import functools

import jax
import jax.numpy as jnp
from jax import lax
from jax.experimental import pallas as pl
from jax.experimental.pallas import tpu as pltpu
from jax.experimental.pallas import tpu_sc as plsc

_DIM = 32
_EPS = 1e-06
_MARGIN = 1.0
_GRP = 16


def _sc_body(bpw, nc, ent_t, rel_t, ph, pr, pt, nh, nr, nt, outp, outn,
             hs, rs, ts, blkh, blkr, blkt, ch, cr, ct, acc, sem, sem2):
    wid = lax.axis_index("s") * nc + lax.axis_index("c")
    base = wid * bpw
    sl = pl.ds(base, bpw)

    def side(h_hbm, r_hbm, t_hbm, out_hbm):
        pltpu.sync_copy(h_hbm.at[sl], hs)
        pltpu.sync_copy(r_hbm.at[sl], rs)
        pltpu.sync_copy(t_hbm.at[sl], ts)

        def group(g, carry):
            gb = g * _GRP
            hv = hs[pl.ds(gb, _GRP)]
            rv = rs[pl.ds(gb, _GRP)]
            tv = ts[pl.ds(gb, _GRP)]
            fetches = []
            for kk in range(_GRP):
                ds_k = pl.ds(kk * _GRP, _GRP)
                for vec, tbl, blk in ((hv, ent_t, blkh), (rv, rel_t, blkr),
                                      (tv, ent_t, blkt)):
                    e = vec[kk]
                    c = pl.multiple_of((e >> 4) << 4, _GRP)
                    fetches.append(pltpu.async_copy(
                        tbl.at[:, pl.ds(c, _GRP)], blk.at[:, ds_k], sem))
            for f in fetches:
                f.wait()
            lane16 = lax.iota(jnp.int32, _GRP) * _GRP
            hc = lane16 + (hv & 15)
            rc = lane16 + (rv & 15)
            tc = lane16 + (tv & 15)

            def dim_step(j, s):
                jj = jnp.full((_GRP,), j, jnp.int32)
                h = plsc.load_gather(blkh, [jj, hc])
                r = plsc.load_gather(blkr, [jj, rc])
                t = plsc.load_gather(blkt, [jj, tc])
                d = h + r - t + _EPS
                return s + d * d

            sq = lax.fori_loop(0, _DIM, dim_step,
                               jnp.zeros((_GRP,), jnp.float32))
            acc[pl.ds(gb, _GRP)] = sq
            return carry

        lax.fori_loop(0, bpw // _GRP, group, 0)
        pltpu.sync_copy(acc, out_hbm.at[sl])

    side(ph, pr, pt, outp)
    side(nh, nr, nt, outn)


@functools.lru_cache(maxsize=None)
def _make_sc_kernel(batch):
    info = plsc.get_sparse_core_info()
    nc, ns = info.num_cores, info.num_subcores
    nw = nc * ns
    assert batch % (nw * _GRP) == 0
    bpw = batch // nw
    mesh = plsc.VectorSubcoreMesh(core_axis_name="c", subcore_axis_name="s")
    return pl.kernel(
        functools.partial(_sc_body, bpw, nc),
        out_type=[jax.ShapeDtypeStruct((batch,), jnp.float32),
                  jax.ShapeDtypeStruct((batch,), jnp.float32)],
        mesh=mesh,
        compiler_params=pltpu.CompilerParams(
            use_tc_tiling_on_sc=False, needs_layout_passes=False),
        scratch_types=(
            [pltpu.VMEM((bpw,), jnp.int32)] * 3
            + [pltpu.VMEM((_DIM, _GRP * _GRP), jnp.float32)] * 3
            + [pltpu.VMEM((_DIM, _GRP), jnp.float32)] * 3
            + [pltpu.VMEM((bpw,), jnp.float32)]
            + [pltpu.SemaphoreType.DMA] * 2
        ),
    )


def _tc_body(batch, p_ref, n_ref, out_ref):
    hinge = jnp.maximum(
        jnp.sqrt(p_ref[...]) - jnp.sqrt(n_ref[...]) + _MARGIN, 0.0)
    out_ref[0, 0] = jnp.sum(hinge) / batch


def kernel(pos_x, neg_x, ent_emb, rel_emb):
    batch = pos_x.shape[0]
    ph, pr, pt = pos_x[:, 0], pos_x[:, 1], pos_x[:, 2]
    nh, nr, nt = neg_x[:, 0], neg_x[:, 1], neg_x[:, 2]
    pos_sq, neg_sq = _make_sc_kernel(batch)(
        ent_emb.T, rel_emb.T, ph, pr, pt, nh, nr, nt)
    out = pl.pallas_call(
        functools.partial(_tc_body, batch),
        out_shape=jax.ShapeDtypeStruct((1, 1), jnp.float32),
        out_specs=pl.BlockSpec(memory_space=pltpu.SMEM),
    )(pos_sq, neg_sq)
    return out[0, 0]


# speedup vs baseline: 5.8410x; 5.8410x over previous
import functools

import jax
import jax.numpy as jnp
from jax import lax
from jax.experimental import pallas as pl
from jax.experimental.pallas import tpu as pltpu
from jax.experimental.pallas import tpu_sc as plsc

_DIM = 32
_EPS = 1e-06
_MARGIN = 1.0


def _sc_body(bpw, nc, ent_hbm, rel_hbm, ph, pr, pt, nh, nr, nt,
             pos_out, neg_out,
             phv, prv, ptv, nhv, nrv, ntv, hp, rp, tp, hn, rn, tn, sem):
    wid = lax.axis_index("s") * nc + lax.axis_index("c")
    base = wid * bpw
    sl = pl.ds(base, bpw)
    pltpu.sync_copy(ph.at[sl], phv)
    pltpu.sync_copy(pr.at[sl], prv)
    pltpu.sync_copy(pt.at[sl], ptv)
    pltpu.sync_copy(nh.at[sl], nhv)
    pltpu.sync_copy(nr.at[sl], nrv)
    pltpu.sync_copy(nt.at[sl], ntv)
    copies = [
        pltpu.async_copy(ent_hbm.at[phv], hp, sem),
        pltpu.async_copy(rel_hbm.at[prv], rp, sem),
        pltpu.async_copy(ent_hbm.at[ptv], tp, sem),
        pltpu.async_copy(ent_hbm.at[nhv], hn, sem),
        pltpu.async_copy(rel_hbm.at[nrv], rn, sem),
        pltpu.async_copy(ent_hbm.at[ntv], tn, sem),
    ]
    for c in copies:
        c.wait()

    def row(i, carry):
        for col in (0, 16):
            csl = pl.ds(col, 16)
            hp[i, csl] = hp[i, csl] + rp[i, csl] - tp[i, csl] + _EPS
            hn[i, csl] = hn[i, csl] + rn[i, csl] - tn[i, csl] + _EPS
        return carry

    lax.fori_loop(0, bpw, row, 0, unroll=4)
    pltpu.sync_copy(hp, pos_out.at[sl, :])
    pltpu.sync_copy(hn, neg_out.at[sl, :])


@functools.lru_cache(maxsize=None)
def _make_sc_gather_diff(batch):
    info = plsc.get_sparse_core_info()
    nc, ns = info.num_cores, info.num_subcores
    nw = nc * ns
    assert batch % nw == 0
    bpw = batch // nw
    mesh = plsc.VectorSubcoreMesh(core_axis_name="c", subcore_axis_name="s")
    return pl.kernel(
        functools.partial(_sc_body, bpw, nc),
        out_type=[
            jax.ShapeDtypeStruct((batch, _DIM), jnp.float32),
            jax.ShapeDtypeStruct((batch, _DIM), jnp.float32),
        ],
        mesh=mesh,
        compiler_params=pltpu.CompilerParams(use_tc_tiling_on_sc=False),
        scratch_types=(
            [pltpu.VMEM((bpw,), jnp.int32)] * 6
            + [pltpu.VMEM((bpw, _DIM), jnp.float32)] * 6
            + [pltpu.SemaphoreType.DMA]
        ),
    )


def _tc_body(pd_ref, nd_ref, out_ref):
    pd = pd_ref[...]
    nd = nd_ref[...]
    ps = jnp.sum(pd * pd, axis=1)
    ns = jnp.sum(nd * nd, axis=1)
    hinge = jnp.maximum(jnp.sqrt(ps) - jnp.sqrt(ns) + _MARGIN, 0.0)
    out_ref[0, 0] = jnp.sum(hinge) / pd.shape[0]


def kernel(pos_x, neg_x, ent_emb, rel_emb):
    batch = pos_x.shape[0]
    ph, pr, pt = pos_x[:, 0], pos_x[:, 1], pos_x[:, 2]
    nh, nr, nt = neg_x[:, 0], neg_x[:, 1], neg_x[:, 2]
    pos_diff, neg_diff = _make_sc_gather_diff(batch)(
        ent_emb, rel_emb, ph, pr, pt, nh, nr, nt)
    out = pl.pallas_call(
        _tc_body,
        out_shape=jax.ShapeDtypeStruct((1, 1), jnp.float32),
        out_specs=pl.BlockSpec(memory_space=pltpu.SMEM),
    )(pos_diff, neg_diff)
    return out[0, 0]
